# Initial kernel scaffold; baseline (speedup 1.0000x reference)
import jax
import jax.numpy as jnp
from jax.experimental import pallas as pl


def kernel(x, target, H, W, Wq, Wk, Wv, Wp, bp):
    raise NotImplementedError("write your pallas kernel here")


# speedup vs baseline: 1.4895x; 1.4895x over previous
import functools

import jax
import jax.numpy as jnp
from jax.experimental import pallas as pl
from jax.experimental.pallas import tpu as pltpu

NH = 8


def _fused_attn_kernel(x_ref, t_ref, wq_ref, wk_ref, wv_ref, wp_ref, bp_ref,
                       out_ref, *, nh, temp):
    x = x_ref[0]
    t = t_ref[0]
    q = jnp.dot(x, wq_ref[:].T, preferred_element_type=jnp.float32)
    k = jnp.dot(t, wk_ref[:].T, preferred_element_type=jnp.float32)
    v = jnp.dot(t, wv_ref[:].T, preferred_element_type=jnp.float32)
    n, c = x.shape
    hd = c // nh
    acc = jnp.broadcast_to(bp_ref[0], (n, c))
    for h in range(nh):
        sl = slice(h * hd, (h + 1) * hd)
        s = jnp.dot(q[:, sl], k[:, sl].T,
                    preferred_element_type=jnp.float32) * temp
        s = s - jnp.max(s, axis=1, keepdims=True)
        p = jnp.exp(s)
        a = p / jnp.sum(p, axis=1, keepdims=True)
        msg = jnp.dot(a, v[:, sl], preferred_element_type=jnp.float32)
        acc = acc + jnp.dot(msg, wp_ref[:, sl].T,
                            preferred_element_type=jnp.float32)
    out_ref[0] = acc


def kernel(x, target, H, W, Wq, Wk, Wv, Wp, bp):
    Bb, Nn, Cc = x.shape
    hd = Cc // NH
    temp = 1.0 / (hd ** 0.5)
    body = functools.partial(_fused_attn_kernel, nh=NH, temp=temp)
    out = pl.pallas_call(
        body,
        grid=(Bb,),
        in_specs=[
            pl.BlockSpec((1, Nn, Cc), lambda b: (b, 0, 0)),
            pl.BlockSpec((1, Nn, Cc), lambda b: (b, 0, 0)),
            pl.BlockSpec((Cc, Cc), lambda b: (0, 0)),
            pl.BlockSpec((Cc, Cc), lambda b: (0, 0)),
            pl.BlockSpec((Cc, Cc), lambda b: (0, 0)),
            pl.BlockSpec((Cc, Cc), lambda b: (0, 0)),
            pl.BlockSpec((1, Cc), lambda b: (0, 0)),
        ],
        out_specs=pl.BlockSpec((1, Nn, Cc), lambda b: (b, 0, 0)),
        out_shape=jax.ShapeDtypeStruct((Bb, Nn, Cc), jnp.float32),
        compiler_params=pltpu.CompilerParams(
            dimension_semantics=("arbitrary",),
        ),
    )(x, target, Wq, Wk, Wv, Wp, bp.reshape(1, Cc))
    return out


# speedup vs baseline: 1.7069x; 1.1460x over previous
import functools

import jax
import jax.numpy as jnp
from jax.experimental import pallas as pl
from jax.experimental.pallas import tpu as pltpu

NH = 8


def _fused_attn_kernel(x_ref, t_ref, wq_ref, wk_ref, wv_ref, wp_ref, bp_ref,
                       out_ref, *, nh, temp):
    x = x_ref[0]
    t = t_ref[0]
    q = jnp.dot(x, wq_ref[:].T, preferred_element_type=jnp.float32) * temp
    k = jnp.dot(t, wk_ref[:].T, preferred_element_type=jnp.float32)
    v = jnp.dot(t, wv_ref[:].T, preferred_element_type=jnp.float32)
    n, c = x.shape
    hd = c // nh
    acc = jnp.broadcast_to(bp_ref[0], (n, c))
    for h in range(nh):
        sl = slice(h * hd, (h + 1) * hd)
        s = jnp.dot(q[:, sl], k[:, sl].T,
                    preferred_element_type=jnp.float32)
        p = jnp.exp(s - jnp.max(s, axis=1, keepdims=True))
        msg = jnp.dot(p, v[:, sl], preferred_element_type=jnp.float32)
        msg = msg / jnp.sum(p, axis=1, keepdims=True)
        acc = acc + jnp.dot(msg, wp_ref[:, sl].T,
                            preferred_element_type=jnp.float32)
    out_ref[0] = acc


def kernel(x, target, H, W, Wq, Wk, Wv, Wp, bp):
    Bb, Nn, Cc = x.shape
    hd = Cc // NH
    temp = 1.0 / (hd ** 0.5)
    body = functools.partial(_fused_attn_kernel, nh=NH, temp=temp)
    out = pl.pallas_call(
        body,
        grid=(Bb,),
        in_specs=[
            pl.BlockSpec((1, Nn, Cc), lambda b: (b, 0, 0)),
            pl.BlockSpec((1, Nn, Cc), lambda b: (b, 0, 0)),
            pl.BlockSpec((Cc, Cc), lambda b: (0, 0)),
            pl.BlockSpec((Cc, Cc), lambda b: (0, 0)),
            pl.BlockSpec((Cc, Cc), lambda b: (0, 0)),
            pl.BlockSpec((Cc, Cc), lambda b: (0, 0)),
            pl.BlockSpec((1, Cc), lambda b: (0, 0)),
        ],
        out_specs=pl.BlockSpec((1, Nn, Cc), lambda b: (b, 0, 0)),
        out_shape=jax.ShapeDtypeStruct((Bb, Nn, Cc), jnp.float32),
        compiler_params=pltpu.CompilerParams(
            dimension_semantics=("arbitrary",),
        ),
    )(x, target, Wq, Wk, Wv, Wp, bp.reshape(1, Cc))
    return out


# speedup vs baseline: 1.7106x; 1.0022x over previous
import functools

import jax
import jax.numpy as jnp
from jax.experimental import pallas as pl
from jax.experimental.pallas import tpu as pltpu

NH = 8


def _fused_attn_kernel(x_ref, t_ref, wq_ref, wk_ref, wv_ref, wp_ref, bp_ref,
                       out_ref, *, nh, temp):
    x = x_ref[0]
    t = t_ref[0]
    q = jnp.dot(x, wq_ref[:].T, preferred_element_type=jnp.float32) * temp
    k = jnp.dot(t, wk_ref[:].T, preferred_element_type=jnp.float32)
    v = jnp.dot(t, wv_ref[:].T, preferred_element_type=jnp.float32)
    n, c = x.shape
    hd = c // nh
    acc = jnp.broadcast_to(bp_ref[0], (n, c))
    for h in range(nh):
        sl = slice(h * hd, (h + 1) * hd)
        s = jnp.dot(q[:, sl], k[:, sl].T,
                    preferred_element_type=jnp.float32)
        p = jnp.exp(s - jnp.max(s, axis=1, keepdims=True))
        msg = jnp.dot(p, v[:, sl], preferred_element_type=jnp.float32)
        msg = msg / jnp.sum(p, axis=1, keepdims=True)
        acc = acc + jnp.dot(msg, wp_ref[:, sl].T,
                            preferred_element_type=jnp.float32)
    out_ref[0] = acc


def kernel(x, target, H, W, Wq, Wk, Wv, Wp, bp):
    Bb, Nn, Cc = x.shape
    hd = Cc // NH
    temp = 1.0 / (hd ** 0.5)
    body = functools.partial(_fused_attn_kernel, nh=NH, temp=temp)
    out = pl.pallas_call(
        body,
        grid=(Bb,),
        in_specs=[
            pl.BlockSpec((1, Nn, Cc), lambda b: (b, 0, 0)),
            pl.BlockSpec((1, Nn, Cc), lambda b: (b, 0, 0)),
            pl.BlockSpec((Cc, Cc), lambda b: (0, 0)),
            pl.BlockSpec((Cc, Cc), lambda b: (0, 0)),
            pl.BlockSpec((Cc, Cc), lambda b: (0, 0)),
            pl.BlockSpec((Cc, Cc), lambda b: (0, 0)),
            pl.BlockSpec((1, Cc), lambda b: (0, 0)),
        ],
        out_specs=pl.BlockSpec((1, Nn, Cc), lambda b: (b, 0, 0)),
        out_shape=jax.ShapeDtypeStruct((Bb, Nn, Cc), jnp.float32),
        compiler_params=pltpu.CompilerParams(
            dimension_semantics=("parallel",),
        ),
    )(x, target, Wq, Wk, Wv, Wp, bp.reshape(1, Cc))
    return out
